# Initial kernel scaffold; baseline (speedup 1.0000x reference)
import jax
import jax.numpy as jnp
from jax.experimental import pallas as pl


def kernel(x_address, x_transaction, edge_a2t, edge_t2a, c1_address_W, c1_address_b, c1_transaction_W, c1_transaction_b, c1_asrc_a2t, c1_adst_a2t, c1_asrc_t2a, c1_adst_t2a, c1_kW, c1_kb, c1_q, c2_address_W, c2_address_b, c2_transaction_W, c2_transaction_b, c2_asrc_a2t, c2_adst_a2t, c2_asrc_t2a, c2_adst_t2a, c2_kW, c2_kb, c2_q, ln1_g, ln1_b, ln2_g, ln2_b, lin_W, lin_b):
    raise NotImplementedError("write your pallas kernel here")


# speedup vs baseline: 34.2249x; 34.2249x over previous
import functools

import jax
import jax.numpy as jnp
import numpy as np
from jax import lax
from jax.experimental import pallas as pl
from jax.experimental.pallas import tpu as pltpu
from jax.experimental.pallas import tpu_sc as plsc

_N = 10000
_NE = 160000
_H = 8
_NC = 2
_NS = 16
_L = 16
_CB = 4000
_BLK = 1000

_SDS = jax.ShapeDtypeStruct


def _sc_mesh():
    return plsc.VectorSubcoreMesh(
        core_axis_name="c", subcore_axis_name="s",
        num_cores=_NC, num_subcores=_NS)


def _sc_edge_logits():
    half = _NE // 2
    nchunk = half // _CB

    def body(ast_ref, adt_ref, gmx_ref, edg_ref, e_ref, sp_ref,
             as_loc, ad_loc, s_loc, sbuf, dbuf, ebuf, gv):
        r = lax.axis_index("c")
        s = lax.axis_index("s")
        h = s // 2
        p = lax.rem(s, 2)
        rh = r * _H + h
        pltpu.sync_copy(ast_ref.at[pl.ds(rh * _N, _N)], as_loc)
        pltpu.sync_copy(adt_ref.at[pl.ds(rh * _N, _N)], ad_loc)
        pltpu.sync_copy(gmx_ref.at[pl.ds(rh * _L, _L)], gv)

        def zb(i, carry):
            s_loc[pl.ds(i * _L, _L)] = jnp.zeros((_L,), jnp.float32)
            return carry
        lax.fori_loop(0, _N // _L, zb, 0)

        gvv = gv[...]
        base = p * half

        def chunk(k, carry):
            off = base + k * _CB
            pltpu.sync_copy(edg_ref.at[pl.ds(r * 2 * _NE + off, _CB)], sbuf)
            pltpu.sync_copy(edg_ref.at[pl.ds((r * 2 + 1) * _NE + off, _CB)], dbuf)

            def grp(i, c2):
                sv = sbuf[pl.ds(i * _L, _L)]
                dv = dbuf[pl.ds(i * _L, _L)]
                av = plsc.load_gather(as_loc, [sv]) + plsc.load_gather(ad_loc, [dv])
                av = jnp.where(av >= 0.0, av, 0.2 * av)
                ev = jnp.exp(av - gvv)
                ebuf[pl.ds(i * _L, _L)] = ev
                plsc.addupdate_scatter(s_loc, [dv], ev)
                return c2
            lax.fori_loop(0, _CB // _L, grp, 0)
            pltpu.sync_copy(ebuf, e_ref.at[pl.ds(rh * _NE + off, _CB)])
            return carry
        lax.fori_loop(0, nchunk, chunk, 0)
        pltpu.sync_copy(s_loc, sp_ref.at[pl.ds(((r * 2 + p) * _H + h) * _N, _N)])

    return pl.kernel(
        body,
        out_type=(_SDS((2 * _H * _NE,), jnp.float32),
                  _SDS((2 * 2 * _H * _N,), jnp.float32)),
        mesh=_sc_mesh(),
        compiler_params=pltpu.CompilerParams(needs_layout_passes=False),
        scratch_types=[
            pltpu.VMEM((_N,), jnp.float32),
            pltpu.VMEM((_N,), jnp.float32),
            pltpu.VMEM((_N,), jnp.float32),
            pltpu.VMEM((_CB,), jnp.int32),
            pltpu.VMEM((_CB,), jnp.int32),
            pltpu.VMEM((_CB,), jnp.float32),
            pltpu.VMEM((_L,), jnp.float32),
        ],
    )


def _sc_messages(C, rel):
    cpt = C // (_NC * _NS)
    D = C // _H
    nchunk = _NE // _CB

    def body(hT_ref, e_ref, sp_ref, edg_ref, out_ref,
             tbl, out_loc, s_loc, tmp, sbuf, dbuf, ebuf):
        c = lax.axis_index("c")
        s = lax.axis_index("s")
        wid = c * _NS + s
        c0 = wid * cpt
        h = c0 // D
        pltpu.sync_copy(hT_ref.at[pl.ds(c0 * _N, cpt * _N)], tbl)
        pltpu.sync_copy(sp_ref.at[pl.ds((rel * 2 * _H + h) * _N, _N)], s_loc)
        pltpu.sync_copy(sp_ref.at[pl.ds(((rel * 2 + 1) * _H + h) * _N, _N)], tmp)

        def addb(i, carry):
            ix = pl.ds(i * _L, _L)
            s_loc[ix] = s_loc[ix] + tmp[ix]
            return carry
        lax.fori_loop(0, _N // _L, addb, 0)

        def zb(i, carry):
            out_loc[pl.ds(i * _L, _L)] = jnp.zeros((_L,), jnp.float32)
            return carry
        lax.fori_loop(0, (cpt * _N) // _L, zb, 0)

        def chunk(k, carry):
            off = k * _CB
            pltpu.sync_copy(edg_ref.at[pl.ds(rel * 2 * _NE + off, _CB)], sbuf)
            pltpu.sync_copy(edg_ref.at[pl.ds((rel * 2 + 1) * _NE + off, _CB)], dbuf)
            pltpu.sync_copy(e_ref.at[pl.ds((rel * _H + h) * _NE + off, _CB)], ebuf)

            def grp(i, c2):
                sv = sbuf[pl.ds(i * _L, _L)]
                dv = dbuf[pl.ds(i * _L, _L)]
                ev = ebuf[pl.ds(i * _L, _L)]
                wv = ev / (plsc.load_gather(s_loc, [dv]) + 1e-16)
                for j in range(cpt):
                    tv = plsc.load_gather(tbl, [sv + j * _N])
                    plsc.addupdate_scatter(out_loc, [dv + j * _N], tv * wv)
                return c2
            lax.fori_loop(0, _CB // _L, grp, 0)
            return carry
        lax.fori_loop(0, nchunk, chunk, 0)
        pltpu.sync_copy(out_loc, out_ref.at[pl.ds(c0 * _N, cpt * _N)])

    return pl.kernel(
        body,
        out_type=_SDS((C * _N,), jnp.float32),
        mesh=_sc_mesh(),
        compiler_params=pltpu.CompilerParams(needs_layout_passes=False),
        scratch_types=[
            pltpu.VMEM((cpt * _N,), jnp.float32),
            pltpu.VMEM((cpt * _N,), jnp.float32),
            pltpu.VMEM((_N,), jnp.float32),
            pltpu.VMEM((_N,), jnp.float32),
            pltpu.VMEM((_CB,), jnp.int32),
            pltpu.VMEM((_CB,), jnp.int32),
            pltpu.VMEM((_CB,), jnp.float32),
        ],
    )


def _tc_proj(x, W, b, avs, avd, seg):
    N, Cin = x.shape
    C = W.shape[1]
    H = seg.shape[1]

    def body(x_ref, w_ref, b_ref, s_ref, d_ref, g_ref, h_ref, as_ref, ad_ref):
        hv = jnp.dot(x_ref[...], w_ref[...],
                     preferred_element_type=jnp.float32) + b_ref[...]
        h_ref[...] = hv
        as_ref[...] = jnp.dot(hv * s_ref[...], g_ref[...],
                              preferred_element_type=jnp.float32)
        ad_ref[...] = jnp.dot(hv * d_ref[...], g_ref[...],
                              preferred_element_type=jnp.float32)

    return pl.pallas_call(
        body,
        grid=(N // _BLK,),
        in_specs=[
            pl.BlockSpec((_BLK, Cin), lambda i: (i, 0)),
            pl.BlockSpec((Cin, C), lambda i: (0, 0)),
            pl.BlockSpec((1, C), lambda i: (0, 0)),
            pl.BlockSpec((1, C), lambda i: (0, 0)),
            pl.BlockSpec((1, C), lambda i: (0, 0)),
            pl.BlockSpec((C, H), lambda i: (0, 0)),
        ],
        out_specs=[
            pl.BlockSpec((_BLK, C), lambda i: (i, 0)),
            pl.BlockSpec((_BLK, H), lambda i: (i, 0)),
            pl.BlockSpec((_BLK, H), lambda i: (i, 0)),
        ],
        out_shape=[_SDS((N, C), jnp.float32),
                   _SDS((N, H), jnp.float32),
                   _SDS((N, H), jnp.float32)],
    )(x, W, b, avs, avd, seg)


def _ln_relu(u, g, b):
    mu = jnp.mean(u, axis=-1, keepdims=True)
    var = jnp.mean((u - mu) ** 2, axis=-1, keepdims=True)
    return jax.nn.relu((u - mu) / jnp.sqrt(var + 1e-5) * g + b)


def _tc_mid(o, lng, lnb, W, b, avs, avd, seg):
    N, Cin = o.shape
    C = W.shape[1]
    H = seg.shape[1]

    def body(o_ref, g_ref, bb_ref, w_ref, b_ref, s_ref, d_ref, gm_ref,
             h_ref, as_ref, ad_ref):
        y = _ln_relu(jax.nn.relu(o_ref[...]), g_ref[...], bb_ref[...])
        hv = jnp.dot(y, w_ref[...], preferred_element_type=jnp.float32) + b_ref[...]
        h_ref[...] = hv
        as_ref[...] = jnp.dot(hv * s_ref[...], gm_ref[...],
                              preferred_element_type=jnp.float32)
        ad_ref[...] = jnp.dot(hv * d_ref[...], gm_ref[...],
                              preferred_element_type=jnp.float32)

    return pl.pallas_call(
        body,
        grid=(N // _BLK,),
        in_specs=[
            pl.BlockSpec((_BLK, Cin), lambda i: (i, 0)),
            pl.BlockSpec((1, Cin), lambda i: (0, 0)),
            pl.BlockSpec((1, Cin), lambda i: (0, 0)),
            pl.BlockSpec((Cin, C), lambda i: (0, 0)),
            pl.BlockSpec((1, C), lambda i: (0, 0)),
            pl.BlockSpec((1, C), lambda i: (0, 0)),
            pl.BlockSpec((1, C), lambda i: (0, 0)),
            pl.BlockSpec((C, H), lambda i: (0, 0)),
        ],
        out_specs=[
            pl.BlockSpec((_BLK, C), lambda i: (i, 0)),
            pl.BlockSpec((_BLK, H), lambda i: (i, 0)),
            pl.BlockSpec((_BLK, H), lambda i: (i, 0)),
        ],
        out_shape=[_SDS((N, C), jnp.float32),
                   _SDS((N, H), jnp.float32),
                   _SDS((N, H), jnp.float32)],
    )(o, lng, lnb, W, b, avs, avd, seg)


def _tc_fin(o, lng, lnb, Wp, bp):
    N, Cin = o.shape
    C = Wp.shape[1]

    def body(o_ref, g_ref, bb_ref, w_ref, b_ref, out_ref):
        y = _ln_relu(jax.nn.relu(o_ref[...]), g_ref[...], bb_ref[...])
        out_ref[...] = jnp.dot(y, w_ref[...],
                               preferred_element_type=jnp.float32) + b_ref[...]

    return pl.pallas_call(
        body,
        grid=(N // _BLK,),
        in_specs=[
            pl.BlockSpec((_BLK, Cin), lambda i: (i, 0)),
            pl.BlockSpec((1, Cin), lambda i: (0, 0)),
            pl.BlockSpec((1, Cin), lambda i: (0, 0)),
            pl.BlockSpec((Cin, C), lambda i: (0, 0)),
            pl.BlockSpec((1, C), lambda i: (0, 0)),
        ],
        out_specs=[pl.BlockSpec((_BLK, C), lambda i: (i, 0))],
        out_shape=[_SDS((N, C), jnp.float32)],
    )(o, lng, lnb, Wp, bp)[0]


def _seg_matrix(C, H):
    D = C // H
    m = np.zeros((C, H), np.float32)
    for h in range(H):
        m[h * D:(h + 1) * D, h] = 1.0
    return jnp.asarray(m)


def _han_layer(h_a, h_t, as_a, ad_a, as_t, ad_t, edges, e_logits, C,
               need_t=True):
    astk = jnp.stack([as_a.T, as_t.T])
    adtk = jnp.stack([ad_t.T, ad_a.T])
    g = jnp.max(astk, axis=2) + jnp.max(adtk, axis=2)
    gmx = jnp.tile(g[:, :, None], (1, 1, _L)).astype(jnp.float32)

    e_vals, s_part = e_logits(astk.reshape(-1), adtk.reshape(-1),
                              gmx.reshape(-1), edges)
    out_a = _sc_messages(C, 1)(h_t.T.reshape(-1), e_vals, s_part,
                               edges).reshape(C, _N)
    if not need_t:
        return out_a.T, None
    out_t = _sc_messages(C, 0)(h_a.T.reshape(-1), e_vals, s_part,
                               edges).reshape(C, _N)
    return out_a.T, out_t.T


def kernel(x_address, x_transaction, edge_a2t, edge_t2a,
           c1_address_W, c1_address_b, c1_transaction_W, c1_transaction_b,
           c1_asrc_a2t, c1_adst_a2t, c1_asrc_t2a, c1_adst_t2a,
           c1_kW, c1_kb, c1_q,
           c2_address_W, c2_address_b, c2_transaction_W, c2_transaction_b,
           c2_asrc_a2t, c2_adst_a2t, c2_asrc_t2a, c2_adst_t2a,
           c2_kW, c2_kb, c2_q,
           ln1_g, ln1_b, ln2_g, ln2_b, lin_W, lin_b):
    C1, C2 = 128, 64
    edges = jnp.stack([edge_a2t, edge_t2a]).astype(jnp.int32).reshape(-1)
    seg1 = _seg_matrix(C1, _H)
    seg2 = _seg_matrix(C2, _H)
    row = lambda v: v.reshape(1, -1).astype(jnp.float32)

    e_logits = _sc_edge_logits()

    h_a, as_a, ad_a = _tc_proj(x_address, c1_address_W, row(c1_address_b),
                               row(c1_asrc_a2t), row(c1_adst_t2a), seg1)
    h_t, as_t, ad_t = _tc_proj(x_transaction, c1_transaction_W,
                               row(c1_transaction_b),
                               row(c1_asrc_t2a), row(c1_adst_a2t), seg1)
    o_a, o_t = _han_layer(h_a, h_t, as_a, ad_a, as_t, ad_t,
                          edges, e_logits, C1)

    h2_a, as2_a, ad2_a = _tc_mid(o_a, row(ln1_g), row(ln1_b),
                                 c2_address_W, row(c2_address_b),
                                 row(c2_asrc_a2t), row(c2_adst_t2a), seg2)
    h2_t, as2_t, ad2_t = _tc_mid(o_t, row(ln1_g), row(ln1_b),
                                 c2_transaction_W, row(c2_transaction_b),
                                 row(c2_asrc_t2a), row(c2_adst_a2t), seg2)
    o2_a, _ = _han_layer(h2_a, h2_t, as2_a, ad2_a, as2_t, ad2_t,
                         edges, e_logits, C2, need_t=False)

    Wp = jnp.zeros((C2, 128), jnp.float32).at[:, :lin_W.shape[1]].set(lin_W)
    bp = jnp.zeros((1, 128), jnp.float32).at[:, :lin_W.shape[1]].set(lin_b)
    out = _tc_fin(o2_a, row(ln2_g), row(ln2_b), Wp, bp)
    return out[:, :lin_W.shape[1]]


# speedup vs baseline: 49.2627x; 1.4394x over previous
import functools

import jax
import jax.numpy as jnp
import numpy as np
from jax import lax
from jax.experimental import pallas as pl
from jax.experimental.pallas import tpu as pltpu
from jax.experimental.pallas import tpu_sc as plsc

_N = 10000
_NE = 160000
_H = 8
_NC = 2
_NS = 16
_L = 16
_UNROLL = 5
_BLK = 1000

_SDS = jax.ShapeDtypeStruct


def _sc_mesh():
    return plsc.VectorSubcoreMesh(
        core_axis_name="c", subcore_axis_name="s",
        num_cores=_NC, num_subcores=_NS)


def _sc_edge_logits():
    half = _NE // 2
    cb = 4000
    nchunk = half // cb
    ng = cb // (_L * _UNROLL)

    def body(ast_ref, adt_ref, gmx_ref, edg_ref, e_ref, sp_ref,
             as_loc, ad_loc, s_loc,
             sb0, db0, eb0, sb1, db1, eb1, gv,
             ss0, sd0, se0, ss1, sd1, se1):
        r = lax.axis_index("c")
        s = lax.axis_index("s")
        h = s // 2
        p = lax.rem(s, 2)
        rh = r * _H + h
        pltpu.sync_copy(ast_ref.at[pl.ds(rh * _N, _N)], as_loc)
        pltpu.sync_copy(adt_ref.at[pl.ds(rh * _N, _N)], ad_loc)
        pltpu.sync_copy(gmx_ref.at[pl.ds(rh * _L, _L)], gv)

        def zb(i, carry):
            for u in range(_UNROLL):
                s_loc[pl.ds((i * _UNROLL + u) * _L, _L)] = jnp.zeros(
                    (_L,), jnp.float32)
            return carry
        lax.fori_loop(0, _N // (_L * _UNROLL), zb, 0)

        gvv = gv[...]
        base = p * half
        slots = ((sb0, db0, eb0, ss0, sd0, se0),
                 (sb1, db1, eb1, ss1, sd1, se1))

        def src_slice(k):
            return edg_ref.at[pl.ds(r * 2 * _NE + base + k * cb, cb)]

        def dst_slice(k):
            return edg_ref.at[pl.ds((r * 2 + 1) * _NE + base + k * cb, cb)]

        def e_slice(k):
            return e_ref.at[pl.ds(rh * _NE + base + k * cb, cb)]

        for b in (0, 1):
            sb, db, eb, s1, s2, s3 = slots[b]
            pltpu.async_copy(src_slice(b), sb, s1)
            pltpu.async_copy(dst_slice(b), db, s2)

        def outer(kk, carry):
            for b in (0, 1):
                k = kk * 2 + b
                sb, db, eb, s1, s2, s3 = slots[b]
                pltpu.make_async_copy(src_slice(k), sb, s1).wait()
                pltpu.make_async_copy(dst_slice(k), db, s2).wait()

                @pl.when(k >= 2)
                def _():
                    pltpu.make_async_copy(eb, e_slice(k - 2), s3).wait()

                def grp(i, c2):
                    for u in range(_UNROLL):
                        ix = pl.ds((i * _UNROLL + u) * _L, _L)
                        sv = sb[ix]
                        dv = db[ix]
                        av = (plsc.load_gather(as_loc, [sv])
                              + plsc.load_gather(ad_loc, [dv]))
                        av = jnp.where(av >= 0.0, av, 0.2 * av)
                        ev = jnp.exp(av - gvv)
                        eb[ix] = ev
                        plsc.addupdate_scatter(s_loc, [dv], ev)
                    return c2
                lax.fori_loop(0, ng, grp, 0)
                pltpu.async_copy(eb, e_slice(k), s3)

                @pl.when(k + 2 < nchunk)
                def _():
                    pltpu.async_copy(src_slice(k + 2), sb, s1)
                    pltpu.async_copy(dst_slice(k + 2), db, s2)
            return carry
        lax.fori_loop(0, nchunk // 2, outer, 0)
        for b in (0, 1):
            sb, db, eb, s1, s2, s3 = slots[b]
            pltpu.make_async_copy(eb, e_slice(nchunk - 2 + b), s3).wait()
        pltpu.sync_copy(s_loc, sp_ref.at[pl.ds(((r * 2 + p) * _H + h) * _N, _N)])

    return pl.kernel(
        body,
        out_type=(_SDS((2 * _H * _NE,), jnp.float32),
                  _SDS((2 * 2 * _H * _N,), jnp.float32)),
        mesh=_sc_mesh(),
        compiler_params=pltpu.CompilerParams(needs_layout_passes=False),
        scratch_types=[
            pltpu.VMEM((_N,), jnp.float32),
            pltpu.VMEM((_N,), jnp.float32),
            pltpu.VMEM((_N,), jnp.float32),
            pltpu.VMEM((cb,), jnp.int32),
            pltpu.VMEM((cb,), jnp.int32),
            pltpu.VMEM((cb,), jnp.float32),
            pltpu.VMEM((cb,), jnp.int32),
            pltpu.VMEM((cb,), jnp.int32),
            pltpu.VMEM((cb,), jnp.float32),
            pltpu.VMEM((_L,), jnp.float32),
            pltpu.SemaphoreType.DMA,
            pltpu.SemaphoreType.DMA,
            pltpu.SemaphoreType.DMA,
            pltpu.SemaphoreType.DMA,
            pltpu.SemaphoreType.DMA,
            pltpu.SemaphoreType.DMA,
        ],
    )


def _sc_messages(C, rel):
    cpt = C // (_NC * _NS)
    D = C // _H
    cb = 3200
    nchunk = _NE // cb
    ng = cb // (_L * _UNROLL)

    def body(hT_ref, e_ref, sp_ref, edg_ref, out_ref,
             tbl, out_loc, s_loc, tmp,
             sb0, db0, eb0, sb1, db1, eb1,
             ss0, sd0, se0, ss1, sd1, se1):
        c = lax.axis_index("c")
        s = lax.axis_index("s")
        wid = c * _NS + s
        c0 = wid * cpt
        h = c0 // D
        pltpu.sync_copy(hT_ref.at[pl.ds(c0 * _N, cpt * _N)], tbl)
        pltpu.sync_copy(sp_ref.at[pl.ds((rel * 2 * _H + h) * _N, _N)], s_loc)
        pltpu.sync_copy(sp_ref.at[pl.ds(((rel * 2 + 1) * _H + h) * _N, _N)], tmp)

        def addb(i, carry):
            for u in range(_UNROLL):
                ix = pl.ds((i * _UNROLL + u) * _L, _L)
                s_loc[ix] = 1.0 / (s_loc[ix] + tmp[ix] + 1e-16)
            return carry
        lax.fori_loop(0, _N // (_L * _UNROLL), addb, 0)

        def zb(i, carry):
            for u in range(_UNROLL):
                out_loc[pl.ds((i * _UNROLL + u) * _L, _L)] = jnp.zeros(
                    (_L,), jnp.float32)
            return carry
        lax.fori_loop(0, (cpt * _N) // (_L * _UNROLL), zb, 0)

        slots = ((sb0, db0, eb0, ss0, sd0, se0),
                 (sb1, db1, eb1, ss1, sd1, se1))

        def src_slice(k):
            return edg_ref.at[pl.ds(rel * 2 * _NE + k * cb, cb)]

        def dst_slice(k):
            return edg_ref.at[pl.ds((rel * 2 + 1) * _NE + k * cb, cb)]

        def e_slice(k):
            return e_ref.at[pl.ds((rel * _H + h) * _NE + k * cb, cb)]

        def issue(k, slot):
            sb, db, eb, s1, s2, s3 = slot
            pltpu.async_copy(src_slice(k), sb, s1)
            pltpu.async_copy(dst_slice(k), db, s2)
            pltpu.async_copy(e_slice(k), eb, s3)

        for b in (0, 1):
            issue(b, slots[b])

        def outer(kk, carry):
            for b in (0, 1):
                k = kk * 2 + b
                sb, db, eb, s1, s2, s3 = slots[b]
                pltpu.make_async_copy(src_slice(k), sb, s1).wait()
                pltpu.make_async_copy(dst_slice(k), db, s2).wait()
                pltpu.make_async_copy(e_slice(k), eb, s3).wait()

                def grp(i, c2):
                    for u in range(_UNROLL):
                        ix = pl.ds((i * _UNROLL + u) * _L, _L)
                        sv = sb[ix]
                        dv = db[ix]
                        ev = eb[ix]
                        wv = ev * plsc.load_gather(s_loc, [dv])
                        for j in range(cpt):
                            tv = plsc.load_gather(tbl, [sv + j * _N])
                            plsc.addupdate_scatter(out_loc, [dv + j * _N],
                                                   tv * wv)
                    return c2
                lax.fori_loop(0, ng, grp, 0)

                @pl.when(k + 2 < nchunk)
                def _():
                    issue(k + 2, slots[b])
            return carry
        lax.fori_loop(0, nchunk // 2, outer, 0)
        pltpu.sync_copy(out_loc, out_ref.at[pl.ds(c0 * _N, cpt * _N)])

    return pl.kernel(
        body,
        out_type=_SDS((C * _N,), jnp.float32),
        mesh=_sc_mesh(),
        compiler_params=pltpu.CompilerParams(needs_layout_passes=False),
        scratch_types=[
            pltpu.VMEM((cpt * _N,), jnp.float32),
            pltpu.VMEM((cpt * _N,), jnp.float32),
            pltpu.VMEM((_N,), jnp.float32),
            pltpu.VMEM((_N,), jnp.float32),
            pltpu.VMEM((cb,), jnp.int32),
            pltpu.VMEM((cb,), jnp.int32),
            pltpu.VMEM((cb,), jnp.float32),
            pltpu.VMEM((cb,), jnp.int32),
            pltpu.VMEM((cb,), jnp.int32),
            pltpu.VMEM((cb,), jnp.float32),
            pltpu.SemaphoreType.DMA,
            pltpu.SemaphoreType.DMA,
            pltpu.SemaphoreType.DMA,
            pltpu.SemaphoreType.DMA,
            pltpu.SemaphoreType.DMA,
            pltpu.SemaphoreType.DMA,
        ],
    )


def _tc_proj(x, W, b, avs, avd, seg):
    N, Cin = x.shape
    C = W.shape[1]
    H = seg.shape[1]

    def body(x_ref, w_ref, b_ref, s_ref, d_ref, g_ref, h_ref, as_ref, ad_ref):
        hv = jnp.dot(x_ref[...], w_ref[...],
                     preferred_element_type=jnp.float32) + b_ref[...]
        h_ref[...] = hv
        as_ref[...] = jnp.dot(hv * s_ref[...], g_ref[...],
                              preferred_element_type=jnp.float32)
        ad_ref[...] = jnp.dot(hv * d_ref[...], g_ref[...],
                              preferred_element_type=jnp.float32)

    return pl.pallas_call(
        body,
        grid=(N // _BLK,),
        in_specs=[
            pl.BlockSpec((_BLK, Cin), lambda i: (i, 0)),
            pl.BlockSpec((Cin, C), lambda i: (0, 0)),
            pl.BlockSpec((1, C), lambda i: (0, 0)),
            pl.BlockSpec((1, C), lambda i: (0, 0)),
            pl.BlockSpec((1, C), lambda i: (0, 0)),
            pl.BlockSpec((C, H), lambda i: (0, 0)),
        ],
        out_specs=[
            pl.BlockSpec((_BLK, C), lambda i: (i, 0)),
            pl.BlockSpec((_BLK, H), lambda i: (i, 0)),
            pl.BlockSpec((_BLK, H), lambda i: (i, 0)),
        ],
        out_shape=[_SDS((N, C), jnp.float32),
                   _SDS((N, H), jnp.float32),
                   _SDS((N, H), jnp.float32)],
    )(x, W, b, avs, avd, seg)


def _ln_relu(u, g, b):
    mu = jnp.mean(u, axis=-1, keepdims=True)
    var = jnp.mean((u - mu) ** 2, axis=-1, keepdims=True)
    return jax.nn.relu((u - mu) / jnp.sqrt(var + 1e-5) * g + b)


def _tc_mid(o, lng, lnb, W, b, avs, avd, seg):
    N, Cin = o.shape
    C = W.shape[1]
    H = seg.shape[1]

    def body(o_ref, g_ref, bb_ref, w_ref, b_ref, s_ref, d_ref, gm_ref,
             h_ref, as_ref, ad_ref):
        y = _ln_relu(jax.nn.relu(o_ref[...]), g_ref[...], bb_ref[...])
        hv = jnp.dot(y, w_ref[...], preferred_element_type=jnp.float32) + b_ref[...]
        h_ref[...] = hv
        as_ref[...] = jnp.dot(hv * s_ref[...], gm_ref[...],
                              preferred_element_type=jnp.float32)
        ad_ref[...] = jnp.dot(hv * d_ref[...], gm_ref[...],
                              preferred_element_type=jnp.float32)

    return pl.pallas_call(
        body,
        grid=(N // _BLK,),
        in_specs=[
            pl.BlockSpec((_BLK, Cin), lambda i: (i, 0)),
            pl.BlockSpec((1, Cin), lambda i: (0, 0)),
            pl.BlockSpec((1, Cin), lambda i: (0, 0)),
            pl.BlockSpec((Cin, C), lambda i: (0, 0)),
            pl.BlockSpec((1, C), lambda i: (0, 0)),
            pl.BlockSpec((1, C), lambda i: (0, 0)),
            pl.BlockSpec((1, C), lambda i: (0, 0)),
            pl.BlockSpec((C, H), lambda i: (0, 0)),
        ],
        out_specs=[
            pl.BlockSpec((_BLK, C), lambda i: (i, 0)),
            pl.BlockSpec((_BLK, H), lambda i: (i, 0)),
            pl.BlockSpec((_BLK, H), lambda i: (i, 0)),
        ],
        out_shape=[_SDS((N, C), jnp.float32),
                   _SDS((N, H), jnp.float32),
                   _SDS((N, H), jnp.float32)],
    )(o, lng, lnb, W, b, avs, avd, seg)


def _tc_fin(o, lng, lnb, Wp, bp):
    N, Cin = o.shape
    C = Wp.shape[1]

    def body(o_ref, g_ref, bb_ref, w_ref, b_ref, out_ref):
        y = _ln_relu(jax.nn.relu(o_ref[...]), g_ref[...], bb_ref[...])
        out_ref[...] = jnp.dot(y, w_ref[...],
                               preferred_element_type=jnp.float32) + b_ref[...]

    return pl.pallas_call(
        body,
        grid=(N // _BLK,),
        in_specs=[
            pl.BlockSpec((_BLK, Cin), lambda i: (i, 0)),
            pl.BlockSpec((1, Cin), lambda i: (0, 0)),
            pl.BlockSpec((1, Cin), lambda i: (0, 0)),
            pl.BlockSpec((Cin, C), lambda i: (0, 0)),
            pl.BlockSpec((1, C), lambda i: (0, 0)),
        ],
        out_specs=[pl.BlockSpec((_BLK, C), lambda i: (i, 0))],
        out_shape=[_SDS((N, C), jnp.float32)],
    )(o, lng, lnb, Wp, bp)[0]


def _seg_matrix(C, H):
    D = C // H
    m = np.zeros((C, H), np.float32)
    for h in range(H):
        m[h * D:(h + 1) * D, h] = 1.0
    return jnp.asarray(m)


def _han_layer(h_a, h_t, as_a, ad_a, as_t, ad_t, edges, e_logits, C,
               need_t=True):
    astk = jnp.stack([as_a.T, as_t.T])
    adtk = jnp.stack([ad_t.T, ad_a.T])
    g = jnp.max(astk, axis=2) + jnp.max(adtk, axis=2)
    gmx = jnp.tile(g[:, :, None], (1, 1, _L)).astype(jnp.float32)

    e_vals, s_part = e_logits(astk.reshape(-1), adtk.reshape(-1),
                              gmx.reshape(-1), edges)
    out_a = _sc_messages(C, 1)(h_t.T.reshape(-1), e_vals, s_part,
                               edges).reshape(C, _N)
    if not need_t:
        return out_a.T, None
    out_t = _sc_messages(C, 0)(h_a.T.reshape(-1), e_vals, s_part,
                               edges).reshape(C, _N)
    return out_a.T, out_t.T


def kernel(x_address, x_transaction, edge_a2t, edge_t2a,
           c1_address_W, c1_address_b, c1_transaction_W, c1_transaction_b,
           c1_asrc_a2t, c1_adst_a2t, c1_asrc_t2a, c1_adst_t2a,
           c1_kW, c1_kb, c1_q,
           c2_address_W, c2_address_b, c2_transaction_W, c2_transaction_b,
           c2_asrc_a2t, c2_adst_a2t, c2_asrc_t2a, c2_adst_t2a,
           c2_kW, c2_kb, c2_q,
           ln1_g, ln1_b, ln2_g, ln2_b, lin_W, lin_b):
    C1, C2 = 128, 64
    edges = jnp.stack([edge_a2t, edge_t2a]).astype(jnp.int32).reshape(-1)
    seg1 = _seg_matrix(C1, _H)
    seg2 = _seg_matrix(C2, _H)
    row = lambda v: v.reshape(1, -1).astype(jnp.float32)

    e_logits = _sc_edge_logits()

    h_a, as_a, ad_a = _tc_proj(x_address, c1_address_W, row(c1_address_b),
                               row(c1_asrc_a2t), row(c1_adst_t2a), seg1)
    h_t, as_t, ad_t = _tc_proj(x_transaction, c1_transaction_W,
                               row(c1_transaction_b),
                               row(c1_asrc_t2a), row(c1_adst_a2t), seg1)
    o_a, o_t = _han_layer(h_a, h_t, as_a, ad_a, as_t, ad_t,
                          edges, e_logits, C1)

    h2_a, as2_a, ad2_a = _tc_mid(o_a, row(ln1_g), row(ln1_b),
                                 c2_address_W, row(c2_address_b),
                                 row(c2_asrc_a2t), row(c2_adst_t2a), seg2)
    h2_t, as2_t, ad2_t = _tc_mid(o_t, row(ln1_g), row(ln1_b),
                                 c2_transaction_W, row(c2_transaction_b),
                                 row(c2_asrc_t2a), row(c2_adst_a2t), seg2)
    o2_a, _ = _han_layer(h2_a, h2_t, as2_a, ad2_a, as2_t, ad2_t,
                         edges, e_logits, C2, need_t=False)

    Wp = jnp.zeros((C2, 128), jnp.float32).at[:, :lin_W.shape[1]].set(lin_W)
    bp = jnp.zeros((1, 128), jnp.float32).at[:, :lin_W.shape[1]].set(lin_b)
    out = _tc_fin(o2_a, row(ln2_g), row(ln2_b), Wp, bp)
    return out[:, :lin_W.shape[1]]


# speedup vs baseline: 175.2290x; 3.5570x over previous
import functools

import jax
import jax.numpy as jnp
import numpy as np
from jax import lax
from jax.experimental import pallas as pl
from jax.experimental.pallas import tpu as pltpu
from jax.experimental.pallas import tpu_sc as plsc

_N = 10000
_NE = 160000
_H = 8
_NC = 2
_NS = 16
_L = 16
_UNROLL = 5
_BLK = 1000

_SDS = jax.ShapeDtypeStruct


def _sc_mesh():
    return plsc.VectorSubcoreMesh(
        core_axis_name="c", subcore_axis_name="s",
        num_cores=_NC, num_subcores=_NS)


def _sc_edge_logits():
    half = _NE // 2
    cb = 4000
    nchunk = half // cb
    ng = cb // (_L * _UNROLL)

    def body(ast_ref, adt_ref, gmx_ref, edg_ref, e_ref, sp_ref,
             as_loc, ad_loc, s_loc,
             sb0, db0, eb0, sb1, db1, eb1, gv,
             ss0, sd0, se0, ss1, sd1, se1):
        r = lax.axis_index("c")
        s = lax.axis_index("s")
        h = s // 2
        p = lax.rem(s, 2)
        rh = r * _H + h
        pltpu.sync_copy(ast_ref.at[pl.ds(rh * _N, _N)], as_loc)
        pltpu.sync_copy(adt_ref.at[pl.ds(rh * _N, _N)], ad_loc)
        pltpu.sync_copy(gmx_ref.at[pl.ds(rh * _L, _L)], gv)

        @functools.partial(plsc.parallel_loop, 0, _N // _L, unroll=_UNROLL)
        def _(i):
            s_loc[pl.ds(i * _L, _L)] = jnp.zeros((_L,), jnp.float32)

        gvv = gv[...]
        base = p * half
        slots = ((sb0, db0, eb0, ss0, sd0, se0),
                 (sb1, db1, eb1, ss1, sd1, se1))

        def src_slice(k):
            return edg_ref.at[pl.ds(r * 2 * _NE + base + k * cb, cb)]

        def dst_slice(k):
            return edg_ref.at[pl.ds((r * 2 + 1) * _NE + base + k * cb, cb)]

        def e_slice(k):
            return e_ref.at[pl.ds(rh * _NE + base + k * cb, cb)]

        for b in (0, 1):
            sb, db, eb, s1, s2, s3 = slots[b]
            pltpu.async_copy(src_slice(b), sb, s1)
            pltpu.async_copy(dst_slice(b), db, s2)

        def outer(kk, carry):
            for b in (0, 1):
                k = kk * 2 + b
                sb, db, eb, s1, s2, s3 = slots[b]
                pltpu.make_async_copy(src_slice(k), sb, s1).wait()
                pltpu.make_async_copy(dst_slice(k), db, s2).wait()

                @pl.when(k >= 2)
                def _():
                    pltpu.make_async_copy(eb, e_slice(k - 2), s3).wait()

                @functools.partial(plsc.parallel_loop, 0, cb // _L,
                                   unroll=_UNROLL)
                def _(i):
                    ix = pl.ds(i * _L, _L)
                    sv = sb[ix]
                    dv = db[ix]
                    av = (plsc.load_gather(as_loc, [sv])
                          + plsc.load_gather(ad_loc, [dv]))
                    av = jnp.where(av >= 0.0, av, 0.2 * av)
                    ev = jnp.exp(av - gvv)
                    eb[ix] = ev
                    plsc.addupdate_scatter(s_loc, [dv], ev)
                pltpu.async_copy(eb, e_slice(k), s3)

                @pl.when(k + 2 < nchunk)
                def _():
                    pltpu.async_copy(src_slice(k + 2), sb, s1)
                    pltpu.async_copy(dst_slice(k + 2), db, s2)
            return carry
        lax.fori_loop(0, nchunk // 2, outer, 0)
        for b in (0, 1):
            sb, db, eb, s1, s2, s3 = slots[b]
            pltpu.make_async_copy(eb, e_slice(nchunk - 2 + b), s3).wait()
        pltpu.sync_copy(s_loc, sp_ref.at[pl.ds(((r * 2 + p) * _H + h) * _N, _N)])

    return pl.kernel(
        body,
        out_type=(_SDS((2 * _H * _NE,), jnp.float32),
                  _SDS((2 * 2 * _H * _N,), jnp.float32)),
        mesh=_sc_mesh(),
        compiler_params=pltpu.CompilerParams(needs_layout_passes=False),
        scratch_types=[
            pltpu.VMEM((_N,), jnp.float32),
            pltpu.VMEM((_N,), jnp.float32),
            pltpu.VMEM((_N,), jnp.float32),
            pltpu.VMEM((cb,), jnp.int32),
            pltpu.VMEM((cb,), jnp.int32),
            pltpu.VMEM((cb,), jnp.float32),
            pltpu.VMEM((cb,), jnp.int32),
            pltpu.VMEM((cb,), jnp.int32),
            pltpu.VMEM((cb,), jnp.float32),
            pltpu.VMEM((_L,), jnp.float32),
            pltpu.SemaphoreType.DMA,
            pltpu.SemaphoreType.DMA,
            pltpu.SemaphoreType.DMA,
            pltpu.SemaphoreType.DMA,
            pltpu.SemaphoreType.DMA,
            pltpu.SemaphoreType.DMA,
        ],
    )


def _sc_messages(C, rel):
    cpt = C // (_NC * _NS)
    D = C // _H
    cb = 3200
    nchunk = _NE // cb
    ng = cb // (_L * _UNROLL)

    def body(hT_ref, e_ref, sp_ref, edg_ref, out_ref,
             tbl, out_loc, s_loc, tmp,
             sb0, db0, eb0, sb1, db1, eb1,
             ss0, sd0, se0, ss1, sd1, se1):
        c = lax.axis_index("c")
        s = lax.axis_index("s")
        wid = c * _NS + s
        c0 = wid * cpt
        h = c0 // D
        pltpu.sync_copy(hT_ref.at[pl.ds(c0 * _N, cpt * _N)], tbl)
        pltpu.sync_copy(sp_ref.at[pl.ds((rel * 2 * _H + h) * _N, _N)], s_loc)
        pltpu.sync_copy(sp_ref.at[pl.ds(((rel * 2 + 1) * _H + h) * _N, _N)], tmp)

        @functools.partial(plsc.parallel_loop, 0, _N // _L, unroll=_UNROLL)
        def _(i):
            ix = pl.ds(i * _L, _L)
            s_loc[ix] = 1.0 / (s_loc[ix] + tmp[ix] + 1e-16)

        @functools.partial(plsc.parallel_loop, 0, (cpt * _N) // _L,
                           unroll=_UNROLL)
        def _(i):
            out_loc[pl.ds(i * _L, _L)] = jnp.zeros((_L,), jnp.float32)

        slots = ((sb0, db0, eb0, ss0, sd0, se0),
                 (sb1, db1, eb1, ss1, sd1, se1))

        def src_slice(k):
            return edg_ref.at[pl.ds(rel * 2 * _NE + k * cb, cb)]

        def dst_slice(k):
            return edg_ref.at[pl.ds((rel * 2 + 1) * _NE + k * cb, cb)]

        def e_slice(k):
            return e_ref.at[pl.ds((rel * _H + h) * _NE + k * cb, cb)]

        def issue(k, slot):
            sb, db, eb, s1, s2, s3 = slot
            pltpu.async_copy(src_slice(k), sb, s1)
            pltpu.async_copy(dst_slice(k), db, s2)
            pltpu.async_copy(e_slice(k), eb, s3)

        for b in (0, 1):
            issue(b, slots[b])

        def outer(kk, carry):
            for b in (0, 1):
                k = kk * 2 + b
                sb, db, eb, s1, s2, s3 = slots[b]
                pltpu.make_async_copy(src_slice(k), sb, s1).wait()
                pltpu.make_async_copy(dst_slice(k), db, s2).wait()
                pltpu.make_async_copy(e_slice(k), eb, s3).wait()

                @functools.partial(plsc.parallel_loop, 0, cb // _L,
                                   unroll=_UNROLL)
                def _(i):
                    ix = pl.ds(i * _L, _L)
                    sv = sb[ix]
                    dv = db[ix]
                    ev = eb[ix]
                    wv = ev * plsc.load_gather(s_loc, [dv])
                    for j in range(cpt):
                        tv = plsc.load_gather(tbl, [sv + j * _N])
                        plsc.addupdate_scatter(out_loc, [dv + j * _N],
                                               tv * wv)

                @pl.when(k + 2 < nchunk)
                def _():
                    issue(k + 2, slots[b])
            return carry
        lax.fori_loop(0, nchunk // 2, outer, 0)
        pltpu.sync_copy(out_loc, out_ref.at[pl.ds(c0 * _N, cpt * _N)])

    return pl.kernel(
        body,
        out_type=_SDS((C * _N,), jnp.float32),
        mesh=_sc_mesh(),
        compiler_params=pltpu.CompilerParams(needs_layout_passes=False),
        scratch_types=[
            pltpu.VMEM((cpt * _N,), jnp.float32),
            pltpu.VMEM((cpt * _N,), jnp.float32),
            pltpu.VMEM((_N,), jnp.float32),
            pltpu.VMEM((_N,), jnp.float32),
            pltpu.VMEM((cb,), jnp.int32),
            pltpu.VMEM((cb,), jnp.int32),
            pltpu.VMEM((cb,), jnp.float32),
            pltpu.VMEM((cb,), jnp.int32),
            pltpu.VMEM((cb,), jnp.int32),
            pltpu.VMEM((cb,), jnp.float32),
            pltpu.SemaphoreType.DMA,
            pltpu.SemaphoreType.DMA,
            pltpu.SemaphoreType.DMA,
            pltpu.SemaphoreType.DMA,
            pltpu.SemaphoreType.DMA,
            pltpu.SemaphoreType.DMA,
        ],
    )


def _tc_proj(x, W, b, avs, avd, seg):
    N, Cin = x.shape
    C = W.shape[1]
    H = seg.shape[1]

    def body(x_ref, w_ref, b_ref, s_ref, d_ref, g_ref, h_ref, as_ref, ad_ref):
        hv = jnp.dot(x_ref[...], w_ref[...],
                     preferred_element_type=jnp.float32) + b_ref[...]
        h_ref[...] = hv
        as_ref[...] = jnp.dot(hv * s_ref[...], g_ref[...],
                              preferred_element_type=jnp.float32)
        ad_ref[...] = jnp.dot(hv * d_ref[...], g_ref[...],
                              preferred_element_type=jnp.float32)

    return pl.pallas_call(
        body,
        grid=(N // _BLK,),
        in_specs=[
            pl.BlockSpec((_BLK, Cin), lambda i: (i, 0)),
            pl.BlockSpec((Cin, C), lambda i: (0, 0)),
            pl.BlockSpec((1, C), lambda i: (0, 0)),
            pl.BlockSpec((1, C), lambda i: (0, 0)),
            pl.BlockSpec((1, C), lambda i: (0, 0)),
            pl.BlockSpec((C, H), lambda i: (0, 0)),
        ],
        out_specs=[
            pl.BlockSpec((_BLK, C), lambda i: (i, 0)),
            pl.BlockSpec((_BLK, H), lambda i: (i, 0)),
            pl.BlockSpec((_BLK, H), lambda i: (i, 0)),
        ],
        out_shape=[_SDS((N, C), jnp.float32),
                   _SDS((N, H), jnp.float32),
                   _SDS((N, H), jnp.float32)],
    )(x, W, b, avs, avd, seg)


def _ln_relu(u, g, b):
    mu = jnp.mean(u, axis=-1, keepdims=True)
    var = jnp.mean((u - mu) ** 2, axis=-1, keepdims=True)
    return jax.nn.relu((u - mu) / jnp.sqrt(var + 1e-5) * g + b)


def _tc_mid(o, lng, lnb, W, b, avs, avd, seg):
    N, Cin = o.shape
    C = W.shape[1]
    H = seg.shape[1]

    def body(o_ref, g_ref, bb_ref, w_ref, b_ref, s_ref, d_ref, gm_ref,
             h_ref, as_ref, ad_ref):
        y = _ln_relu(jax.nn.relu(o_ref[...]), g_ref[...], bb_ref[...])
        hv = jnp.dot(y, w_ref[...], preferred_element_type=jnp.float32) + b_ref[...]
        h_ref[...] = hv
        as_ref[...] = jnp.dot(hv * s_ref[...], gm_ref[...],
                              preferred_element_type=jnp.float32)
        ad_ref[...] = jnp.dot(hv * d_ref[...], gm_ref[...],
                              preferred_element_type=jnp.float32)

    return pl.pallas_call(
        body,
        grid=(N // _BLK,),
        in_specs=[
            pl.BlockSpec((_BLK, Cin), lambda i: (i, 0)),
            pl.BlockSpec((1, Cin), lambda i: (0, 0)),
            pl.BlockSpec((1, Cin), lambda i: (0, 0)),
            pl.BlockSpec((Cin, C), lambda i: (0, 0)),
            pl.BlockSpec((1, C), lambda i: (0, 0)),
            pl.BlockSpec((1, C), lambda i: (0, 0)),
            pl.BlockSpec((1, C), lambda i: (0, 0)),
            pl.BlockSpec((C, H), lambda i: (0, 0)),
        ],
        out_specs=[
            pl.BlockSpec((_BLK, C), lambda i: (i, 0)),
            pl.BlockSpec((_BLK, H), lambda i: (i, 0)),
            pl.BlockSpec((_BLK, H), lambda i: (i, 0)),
        ],
        out_shape=[_SDS((N, C), jnp.float32),
                   _SDS((N, H), jnp.float32),
                   _SDS((N, H), jnp.float32)],
    )(o, lng, lnb, W, b, avs, avd, seg)


def _tc_fin(o, lng, lnb, Wp, bp):
    N, Cin = o.shape
    C = Wp.shape[1]

    def body(o_ref, g_ref, bb_ref, w_ref, b_ref, out_ref):
        y = _ln_relu(jax.nn.relu(o_ref[...]), g_ref[...], bb_ref[...])
        out_ref[...] = jnp.dot(y, w_ref[...],
                               preferred_element_type=jnp.float32) + b_ref[...]

    return pl.pallas_call(
        body,
        grid=(N // _BLK,),
        in_specs=[
            pl.BlockSpec((_BLK, Cin), lambda i: (i, 0)),
            pl.BlockSpec((1, Cin), lambda i: (0, 0)),
            pl.BlockSpec((1, Cin), lambda i: (0, 0)),
            pl.BlockSpec((Cin, C), lambda i: (0, 0)),
            pl.BlockSpec((1, C), lambda i: (0, 0)),
        ],
        out_specs=[pl.BlockSpec((_BLK, C), lambda i: (i, 0))],
        out_shape=[_SDS((N, C), jnp.float32)],
    )(o, lng, lnb, Wp, bp)[0]


def _seg_matrix(C, H):
    D = C // H
    m = np.zeros((C, H), np.float32)
    for h in range(H):
        m[h * D:(h + 1) * D, h] = 1.0
    return jnp.asarray(m)


def _han_layer(h_a, h_t, as_a, ad_a, as_t, ad_t, edges, e_logits, C,
               need_t=True):
    astk = jnp.stack([as_a.T, as_t.T])
    adtk = jnp.stack([ad_t.T, ad_a.T])
    g = jnp.max(astk, axis=2) + jnp.max(adtk, axis=2)
    gmx = jnp.tile(g[:, :, None], (1, 1, _L)).astype(jnp.float32)

    e_vals, s_part = e_logits(astk.reshape(-1), adtk.reshape(-1),
                              gmx.reshape(-1), edges)
    out_a = _sc_messages(C, 1)(h_t.T.reshape(-1), e_vals, s_part,
                               edges).reshape(C, _N)
    if not need_t:
        return out_a.T, None
    out_t = _sc_messages(C, 0)(h_a.T.reshape(-1), e_vals, s_part,
                               edges).reshape(C, _N)
    return out_a.T, out_t.T


def kernel(x_address, x_transaction, edge_a2t, edge_t2a,
           c1_address_W, c1_address_b, c1_transaction_W, c1_transaction_b,
           c1_asrc_a2t, c1_adst_a2t, c1_asrc_t2a, c1_adst_t2a,
           c1_kW, c1_kb, c1_q,
           c2_address_W, c2_address_b, c2_transaction_W, c2_transaction_b,
           c2_asrc_a2t, c2_adst_a2t, c2_asrc_t2a, c2_adst_t2a,
           c2_kW, c2_kb, c2_q,
           ln1_g, ln1_b, ln2_g, ln2_b, lin_W, lin_b):
    C1, C2 = 128, 64
    edges = jnp.stack([edge_a2t, edge_t2a]).astype(jnp.int32).reshape(-1)
    seg1 = _seg_matrix(C1, _H)
    seg2 = _seg_matrix(C2, _H)
    row = lambda v: v.reshape(1, -1).astype(jnp.float32)

    e_logits = _sc_edge_logits()

    h_a, as_a, ad_a = _tc_proj(x_address, c1_address_W, row(c1_address_b),
                               row(c1_asrc_a2t), row(c1_adst_t2a), seg1)
    h_t, as_t, ad_t = _tc_proj(x_transaction, c1_transaction_W,
                               row(c1_transaction_b),
                               row(c1_asrc_t2a), row(c1_adst_a2t), seg1)
    o_a, o_t = _han_layer(h_a, h_t, as_a, ad_a, as_t, ad_t,
                          edges, e_logits, C1)

    h2_a, as2_a, ad2_a = _tc_mid(o_a, row(ln1_g), row(ln1_b),
                                 c2_address_W, row(c2_address_b),
                                 row(c2_asrc_a2t), row(c2_adst_t2a), seg2)
    h2_t, as2_t, ad2_t = _tc_mid(o_t, row(ln1_g), row(ln1_b),
                                 c2_transaction_W, row(c2_transaction_b),
                                 row(c2_asrc_t2a), row(c2_adst_a2t), seg2)
    o2_a, _ = _han_layer(h2_a, h2_t, as2_a, ad2_a, as2_t, ad2_t,
                         edges, e_logits, C2, need_t=False)

    Wp = jnp.zeros((C2, 128), jnp.float32).at[:, :lin_W.shape[1]].set(lin_W)
    bp = jnp.zeros((1, 128), jnp.float32).at[:, :lin_W.shape[1]].set(lin_b)
    out = _tc_fin(o2_a, row(ln2_g), row(ln2_b), Wp, bp)
    return out[:, :lin_W.shape[1]]
